# baseline (speedup 1.0000x reference)
import functools

import jax
import jax.numpy as jnp
from jax import lax
from jax.experimental import pallas as pl
from jax.experimental.pallas import tpu as pltpu
from jax.experimental.pallas import tpu_sc as plsc

VIDEO_DIM = 1024
EMBED = 512
VOCAB_ROWS = 1000000
BATCH = 4096
HIST = 50
LANES = 16
NC = 2
NS = 16
NW = NC * NS
BPW = BATCH // NW
NVEC = EMBED // LANES
INV_HIST = 1.0 / HIST


@functools.partial(
    pl.kernel,
    out_type=jax.ShapeDtypeStruct((BATCH, EMBED), jnp.float32),
    mesh=plsc.VectorSubcoreMesh(core_axis_name="c", subcore_axis_name="s"),
    scratch_types=[
        pltpu.VMEM((BPW, HIST), jnp.int32),
        pltpu.VMEM((HIST, EMBED // 128, 128), jnp.float32),
        pltpu.VMEM((HIST, EMBED // 128, 128), jnp.float32),
        pltpu.VMEM((BPW // 2, EMBED), jnp.float32),
        pltpu.SemaphoreType.DMA,
        pltpu.SemaphoreType.DMA,
    ],
)
def _bag_kernel(table_hbm, idx_hbm, out_hbm, idx_v, rows0, rows1, out_v,
                sem0, sem1):
    wid = lax.axis_index("s") * NC + lax.axis_index("c")
    base = wid * BPW

    pltpu.sync_copy(idx_hbm.at[pl.ds(base, BPW)], idx_v)

    def start_gather(b, buf, sem):
        pltpu.async_copy(table_hbm.at[idx_v.at[b]], buf, sem)

    def wait_gather(buf, sem):
        pltpu.make_async_copy(table_hbm.at[idx_v.at[0]], buf, sem).wait()

    start_gather(0, rows0, sem0)
    start_gather(1, rows1, sem1)

    half = BPW // 2
    for h in (0, 1):
        hbase = h * half

        @pl.loop(hbase, hbase + half, step=2)
        def _(b):
            for off, buf, sem in ((0, rows0, sem0), (1, rows1, sem1)):
                bb = b + off
                wait_gather(buf, sem)

                @pl.loop(0, NVEC)
                def _(j):
                    col = j * LANES
                    s = col // 128
                    e = col % 128
                    acc = buf[0, s, pl.ds(e, LANES)]
                    for i in range(1, HIST):
                        acc = acc + buf[i, s, pl.ds(e, LANES)]
                    out_v[bb - hbase, pl.ds(col, LANES)] = acc * INV_HIST

                @pl.when(bb + 2 < BPW)
                def _():
                    start_gather(bb + 2, buf, sem)

        pltpu.sync_copy(out_v, out_hbm.at[pl.ds(base + hbase, half)])


def _mm_body(v_ref, w_ref, b_ref, o_ref):
    o_ref[...] = (
        jnp.dot(v_ref[...], w_ref[...], preferred_element_type=jnp.float32,
                precision=lax.Precision.HIGHEST)
        + b_ref[...]
    )


def _video_embed(video, W, b):
    TM = 512
    return pl.pallas_call(
        _mm_body,
        grid=(BATCH // TM,),
        in_specs=[
            pl.BlockSpec((TM, VIDEO_DIM), lambda i: (i, 0)),
            pl.BlockSpec((VIDEO_DIM, EMBED), lambda i: (0, 0)),
            pl.BlockSpec((1, EMBED), lambda i: (0, 0)),
        ],
        out_specs=pl.BlockSpec((TM, EMBED), lambda i: (i, 0)),
        out_shape=jax.ShapeDtypeStruct((BATCH, EMBED), jnp.float32),
    )(video, W, b.reshape(1, EMBED))


def kernel(video, text, W, b, table):
    idx = text.astype(jnp.int32)
    text_embed = _bag_kernel(table.reshape(VOCAB_ROWS, EMBED // 128, 128), idx)
    video_embed = _video_embed(video, W, b)
    return jnp.concatenate([video_embed, text_embed], axis=-1)


# speedup vs baseline: 5.0923x; 5.0923x over previous
import functools

import jax
import jax.numpy as jnp
from jax import lax
from jax.experimental import pallas as pl
from jax.experimental.pallas import tpu as pltpu
from jax.experimental.pallas import tpu_sc as plsc

VIDEO_DIM = 1024
EMBED = 512
BATCH = 4096
HIST = 50
LANES = 16
SEG = 128
NSEG = EMBED // SEG
NC = 2
NS = 16
NW = NC * NS
BPW = BATCH // NW
INV_HIST = 1.0 / HIST

_ROW_BUFS = [pltpu.VMEM((HIST, SEG), jnp.float32) for _ in range(2 * NSEG)]


@functools.partial(
    pl.kernel,
    out_type=jax.ShapeDtypeStruct((BATCH, EMBED), jnp.float32),
    mesh=plsc.VectorSubcoreMesh(core_axis_name="c", subcore_axis_name="s"),
    scratch_types=[
        pltpu.VMEM((BPW, HIST), jnp.int32),
        *_ROW_BUFS,
        pltpu.VMEM((BPW // 2, EMBED), jnp.float32),
        pltpu.SemaphoreType.DMA,
        pltpu.SemaphoreType.DMA,
    ],
)
def _bag_kernel(table_hbm, idx_hbm, out_hbm, idx_v,
                a0, a1, a2, a3, b0, b1, b2, b3, out_v, sem0, sem1):
    bufs = ((a0, a1, a2, a3), (b0, b1, b2, b3))
    sems = (sem0, sem1)
    wid = lax.axis_index("s") * NC + lax.axis_index("c")
    base = wid * BPW

    pltpu.sync_copy(idx_hbm.at[pl.ds(base, BPW)], idx_v)

    def start_gathers(b, par):
        for s in range(NSEG):
            pltpu.async_copy(
                table_hbm.at[idx_v.at[b], pl.ds(s * SEG, SEG)],
                bufs[par][s], sems[par])

    def wait_gathers(par):
        for s in range(NSEG):
            pltpu.make_async_copy(
                table_hbm.at[idx_v.at[0], pl.ds(0, SEG)],
                bufs[par][s], sems[par]).wait()

    start_gathers(0, 0)
    start_gathers(1, 1)

    half = BPW // 2
    for h in (0, 1):
        hbase = h * half

        @pl.loop(hbase, hbase + half, step=2)
        def _(b):
            for par in (0, 1):
                bb = b + par
                wait_gathers(par)

                for s in range(NSEG):
                    buf = bufs[par][s]

                    @pl.loop(0, SEG // LANES)
                    def _(j):
                        e = j * LANES
                        acc = buf[0, pl.ds(e, LANES)]
                        for i in range(1, HIST):
                            acc = acc + buf[i, pl.ds(e, LANES)]
                        out_v[bb - hbase, pl.ds(s * SEG + e, LANES)] = (
                            acc * INV_HIST)

                @pl.when(bb + 2 < BPW)
                def _():
                    start_gathers(bb + 2, par)

        pltpu.sync_copy(out_v, out_hbm.at[pl.ds(base + hbase, half)])


def _mm_body(v_ref, w_ref, b_ref, o_ref):
    o_ref[...] = (
        jnp.dot(v_ref[...], w_ref[...], preferred_element_type=jnp.float32,
                precision=lax.Precision.HIGHEST)
        + b_ref[...]
    )


def _video_embed(video, W, b):
    TM = 512
    return pl.pallas_call(
        _mm_body,
        grid=(BATCH // TM,),
        in_specs=[
            pl.BlockSpec((TM, VIDEO_DIM), lambda i: (i, 0)),
            pl.BlockSpec((VIDEO_DIM, EMBED), lambda i: (0, 0)),
            pl.BlockSpec((1, EMBED), lambda i: (0, 0)),
        ],
        out_specs=pl.BlockSpec((TM, EMBED), lambda i: (i, 0)),
        out_shape=jax.ShapeDtypeStruct((BATCH, EMBED), jnp.float32),
    )(video, W, b.reshape(1, EMBED))


def kernel(video, text, W, b, table):
    idx = text.astype(jnp.int32)
    text_embed = _bag_kernel(table, idx)
    video_embed = _video_embed(video, W, b)
    return jnp.concatenate([video_embed, text_embed], axis=-1)
